# Initial kernel scaffold; baseline (speedup 1.0000x reference)
import jax
import jax.numpy as jnp
from jax.experimental import pallas as pl


def kernel(inputs, targets, centers):
    raise NotImplementedError("write your pallas kernel here")


# speedup vs baseline: 1.6713x; 1.6713x over previous
import functools

import jax
import jax.numpy as jnp
from jax import lax
from jax.experimental import pallas as pl
from jax.experimental.pallas import tpu as pltpu
from jax.experimental.pallas import tpu_sc as plsc

BATCH = 16384
FEAT = 2048
NUM_CORES = 2
NUM_SUBCORES = 16
NW = NUM_CORES * NUM_SUBCORES
ROWS_PER_W = BATCH // NW
CHUNK = 8
NCHUNK = ROWS_PER_W // CHUNK
LANES = 16
BLK = FEAT // LANES
UNROLL = 8


def _chunk_sum(xb, gb, acc):
    def row_body(r, acc):
        def blk_body(j, acc):
            a = acc
            for u in range(UNROLL):
                off = (j * UNROLL + u) * LANES
                xv = xb[r, pl.ds(off, LANES)]
                gv = gb[r, pl.ds(off, LANES)]
                d = xv - gv
                a = a + d * d
            return a

        return lax.fori_loop(0, BLK // UNROLL, blk_body, acc)

    return lax.fori_loop(0, CHUNK, row_body, acc)


def _make_body():
    mesh = plsc.VectorSubcoreMesh(core_axis_name="c", subcore_axis_name="s")

    @functools.partial(
        pl.kernel,
        out_type=jax.ShapeDtypeStruct((NW, LANES), jnp.float32),
        mesh=mesh,
        scratch_types=[
            pltpu.VMEM((ROWS_PER_W,), jnp.int32),
            pltpu.VMEM((CHUNK, FEAT), jnp.float32),
            pltpu.VMEM((CHUNK, FEAT), jnp.float32),
            pltpu.VMEM((CHUNK, FEAT), jnp.float32),
            pltpu.VMEM((CHUNK, FEAT), jnp.float32),
            pltpu.VMEM((LANES,), jnp.float32),
            pltpu.SemaphoreType.DMA,
            pltpu.SemaphoreType.DMA,
            pltpu.SemaphoreType.DMA,
            pltpu.SemaphoreType.DMA,
        ],
    )
    def body(x_hbm, t_hbm, c_hbm, out_hbm,
             idx_v, xb0, xb1, gb0, gb1, accv,
             sx0, sx1, sg0, sg1):
        wid = lax.axis_index("s") * NUM_CORES + lax.axis_index("c")
        base = wid * ROWS_PER_W

        pltpu.sync_copy(t_hbm.at[pl.ds(base, ROWS_PER_W)], idx_v)

        slots = ((xb0, gb0, sx0, sg0), (xb1, gb1, sx1, sg1))

        def issue(ci, slot):
            xb, gb, sx, sg = slots[slot]
            pltpu.async_copy(
                x_hbm.at[pl.ds(base + ci * CHUNK, CHUNK)], xb, sx)
            pltpu.async_copy(
                c_hbm.at[idx_v.at[pl.ds(ci * CHUNK, CHUNK)]], gb, sg)

        def wait(slot):
            xb, gb, sx, sg = slots[slot]
            pltpu.make_async_copy(x_hbm.at[pl.ds(0, CHUNK)], xb, sx).wait()
            pltpu.make_async_copy(
                c_hbm.at[idx_v.at[pl.ds(0, CHUNK)]], gb, sg).wait()

        issue(0, 0)
        issue(1, 1)

        def pair_body(p, acc):
            for b in range(2):
                ci = p * 2 + b
                wait(b)
                acc = _chunk_sum(slots[b][0], slots[b][1], acc)
                issue(ci + 2, b)
            return acc

        acc = jnp.zeros((LANES,), jnp.float32)
        acc = lax.fori_loop(0, NCHUNK // 2 - 1, pair_body, acc)
        for b in range(2):
            wait(b)
            acc = _chunk_sum(slots[b][0], slots[b][1], acc)

        accv[...] = acc
        pltpu.sync_copy(accv, out_hbm.at[wid])

    return body


_sc_loss = _make_body()


@jax.jit
def kernel(inputs, targets, centers):
    partials = _sc_loss(inputs, targets.astype(jnp.int32), centers)
    return jnp.sum(partials) / inputs.shape[0]


# speedup vs baseline: 1.9347x; 1.1576x over previous
import functools

import jax
import jax.numpy as jnp
from jax import lax
from jax.experimental import pallas as pl
from jax.experimental.pallas import tpu as pltpu
from jax.experimental.pallas import tpu_sc as plsc

BATCH = 16384
FEAT = 2048
NUM_CORES = 2
NUM_SUBCORES = 16
NW = NUM_CORES * NUM_SUBCORES
ROWS_PER_W = BATCH // NW
CHUNK = 8
NCHUNK = ROWS_PER_W // CHUNK
LANES = 16
BLK = FEAT // LANES
UNROLL = 16
NBUF = 3


def _chunk_sum(xb, gb, acc):
    def row_body(r, acc):
        def blk_body(j, acc):
            a = acc
            for u in range(UNROLL):
                off = (j * UNROLL + u) * LANES
                xv = xb[r, pl.ds(off, LANES)]
                gv = gb[r, pl.ds(off, LANES)]
                d = xv - gv
                a = a + d * d
            return a

        return lax.fori_loop(0, BLK // UNROLL, blk_body, acc)

    return lax.fori_loop(0, CHUNK, row_body, acc)


def _make_body():
    mesh = plsc.VectorSubcoreMesh(core_axis_name="c", subcore_axis_name="s")

    @functools.partial(
        pl.kernel,
        out_type=jax.ShapeDtypeStruct((NW, LANES), jnp.float32),
        mesh=mesh,
        scratch_types=(
            [pltpu.VMEM((ROWS_PER_W,), jnp.int32)]
            + [pltpu.VMEM((CHUNK, FEAT), jnp.float32)
               for _ in range(NBUF)]
            + [pltpu.VMEM((CHUNK, FEAT), jnp.float32)
               for _ in range(NBUF)]
            + [pltpu.VMEM((LANES,), jnp.float32)]
            + [pltpu.SemaphoreType.DMA for _ in range(2 * NBUF)]
        ),
    )
    def body(x_hbm, t_hbm, c_hbm, out_hbm, idx_v, *rest):
        xbs = rest[0:NBUF]
        gbs = rest[NBUF:2 * NBUF]
        accv = rest[2 * NBUF]
        sxs = rest[2 * NBUF + 1: 3 * NBUF + 1]
        sgs = rest[3 * NBUF + 1: 4 * NBUF + 1]

        wid = lax.axis_index("s") * NUM_CORES + lax.axis_index("c")
        base = wid * ROWS_PER_W

        pltpu.sync_copy(t_hbm.at[pl.ds(base, ROWS_PER_W)], idx_v)

        slots = tuple(
            (xbs[b], gbs[b], sxs[b], sgs[b]) for b in range(NBUF))

        def issue(ci, slot):
            xb, gb, sx, sg = slots[slot]
            pltpu.async_copy(
                x_hbm.at[pl.ds(base + ci * CHUNK, CHUNK)], xb, sx)
            pltpu.async_copy(
                c_hbm.at[idx_v.at[pl.ds(ci * CHUNK, CHUNK)]], gb, sg)

        def wait(slot):
            xb, gb, sx, sg = slots[slot]
            pltpu.make_async_copy(x_hbm.at[pl.ds(0, CHUNK)], xb, sx).wait()
            pltpu.make_async_copy(
                c_hbm.at[idx_v.at[pl.ds(0, CHUNK)]], gb, sg).wait()

        for b in range(NBUF):
            issue(b, b)

        def ring_body(p, acc):
            for b in range(NBUF):
                ci = p * NBUF + b
                wait(b)
                acc = _chunk_sum(slots[b][0], slots[b][1], acc)
                issue(ci + NBUF, b)
            return acc

        NFULL = (NCHUNK // NBUF - 1) * NBUF
        acc = jnp.zeros((LANES,), jnp.float32)
        acc = lax.fori_loop(0, NCHUNK // NBUF - 1, ring_body, acc)
        for ci in range(NFULL, NCHUNK):
            b = ci % NBUF
            wait(b)
            acc = _chunk_sum(slots[b][0], slots[b][1], acc)
            if ci + NBUF < NCHUNK:
                issue(ci + NBUF, b)

        accv[...] = acc
        pltpu.sync_copy(accv, out_hbm.at[wid])

    return body


_sc_loss = _make_body()


@jax.jit
def kernel(inputs, targets, centers):
    partials = _sc_loss(inputs, targets.astype(jnp.int32), centers)
    return jnp.sum(partials) / inputs.shape[0]
